# baseline (speedup 1.0000x reference)
import functools

import jax
import jax.numpy as jnp
from jax import lax
from jax.experimental import pallas as pl
from jax.experimental.pallas import tpu as pltpu
from jax.experimental.pallas import tpu_sc as plsc

N_NODES = 16384
N_EDGES = 16384
EMBED_DIM = 32
LANES = 16


@functools.lru_cache(maxsize=None)
def _build_mf_kernel():
    info = plsc.get_sparse_core_info()
    nc, ns = info.num_cores, info.num_subcores
    nw = nc * ns
    epw = N_EDGES // nw
    n_chunks = epw // 128

    mesh = plsc.VectorSubcoreMesh(core_axis_name="c", subcore_axis_name="s")

    @functools.partial(
        pl.kernel,
        mesh=mesh,
        out_type=(
            jax.ShapeDtypeStruct((N_EDGES,), jnp.float32),
            jax.ShapeDtypeStruct((N_EDGES,), jnp.float32),
        ),
        scratch_types=[
            pltpu.VMEM((N_NODES,), jnp.int32),
            pltpu.VMEM((N_NODES,), jnp.int32),
            pltpu.VMEM((epw,), jnp.int32),
            pltpu.VMEM((epw,), jnp.int32),
            pltpu.VMEM((n_chunks, 128), jnp.int32),
            pltpu.VMEM((n_chunks, 128), jnp.int32),
            pltpu.VMEM((epw, EMBED_DIM), jnp.float32),
            pltpu.VMEM((epw, EMBED_DIM), jnp.float32),
            pltpu.VMEM((epw,), jnp.float32),
            pltpu.SemaphoreType.DMA,
        ],
        compiler_params=pltpu.CompilerParams(
            needs_layout_passes=False, use_tc_tiling_on_sc=False),
    )
    def mf(ut_ref, it_ref, uidx_ref, iidx_ref, ps_ref, pd_ref, nsrc_ref,
           nd_ref, pos_out, neg_out, uidx_v, iidx_v, src_v, dst_v,
           gid_u, gid_i, urows, irows, scores_v, sem):
        wid = lax.axis_index("s") * nc + lax.axis_index("c")
        base = wid * epw
        iota16 = lax.iota(jnp.int32, LANES)

        pltpu.sync_copy(uidx_ref, uidx_v)
        pltpu.sync_copy(iidx_ref, iidx_v)

        for s_hbm, d_hbm, o_hbm in ((ps_ref, pd_ref, pos_out),
                                    (nsrc_ref, nd_ref, neg_out)):
            pltpu.sync_copy(s_hbm.at[pl.ds(base, epw)], src_v)
            pltpu.sync_copy(d_hbm.at[pl.ds(base, epw)], dst_v)

            for c in range(epw // LANES):
                s = src_v[pl.ds(c * LANES, LANES)]
                d = dst_v[pl.ds(c * LANES, LANES)]
                gu = plsc.load_gather(uidx_v, [s])
                gi = plsc.load_gather(iidx_v, [d])
                r, off = divmod(c * LANES, 128)
                gid_u[r, pl.ds(off, LANES)] = gu
                gid_i[r, pl.ds(off, LANES)] = gi

            copies = []
            for j in range(n_chunks):
                copies.append(pltpu.async_copy(
                    ut_ref.at[gid_u.at[j]],
                    urows.at[pl.ds(j * 128, 128)], sem))
                copies.append(pltpu.async_copy(
                    it_ref.at[gid_i.at[j]],
                    irows.at[pl.ds(j * 128, 128)], sem))
            for cp in copies:
                cp.wait()

            def group_body(g, carry):
                row = g * LANES + iota16
                acc = jnp.zeros((LANES,), jnp.float32)
                for dcol in range(EMBED_DIM):
                    col = jnp.full((LANES,), dcol, jnp.int32)
                    ucol = plsc.load_gather(urows, [row, col])
                    icol = plsc.load_gather(irows, [row, col])
                    acc = acc + ucol * icol
                plsc.store_scatter(scores_v, [row], acc)
                return carry

            lax.fori_loop(0, epw // LANES, group_body, 0)
            pltpu.sync_copy(scores_v, o_hbm.at[pl.ds(base, epw)])

    return mf


def kernel(user_table, item_table, user_idx, item_idx, pos_src, pos_dst,
           neg_src, neg_dst):
    mf = _build_mf_kernel()
    pos, neg = mf(
        user_table, item_table,
        user_idx.astype(jnp.int32), item_idx.astype(jnp.int32),
        pos_src.astype(jnp.int32), pos_dst.astype(jnp.int32),
        neg_src.astype(jnp.int32), neg_dst.astype(jnp.int32),
    )
    return pos.reshape(N_EDGES, 1), neg.reshape(N_EDGES, 1)
